# baseline (speedup 1.0000x reference)
import functools

import jax
import jax.numpy as jnp
from jax import lax
from jax.experimental import pallas as pl
from jax.experimental.pallas import tpu as pltpu
from jax.experimental.pallas import tpu_sc as plsc


def _fused_body(mask_ref, rw_ref, b_ref, x_ref, w_ref, logits_ref, y_ref,
                pooled_acc, cnt_ref, *, bb, tsteps, npool, E, C, CP):
    i = pl.program_id(0)

    @pl.when(i < npool)
    def _pool():
        b = i // tsteps
        t = i % tsteps
        mask_f = mask_ref[...].astype(jnp.float32)
        part = jnp.sum(x_ref[...] * mask_f[:, :, None], axis=1)
        cnt = jnp.sum(mask_f, axis=1, keepdims=True)
        rows = pl.ds(b * bb, bb)

        @pl.when(t == 0)
        def _init():
            pooled_acc[rows, :] = part
            cnt_ref[...] = cnt + jnp.zeros_like(cnt_ref)

        @pl.when(t > 0)
        def _accum():
            pooled_acc[rows, :] += part
            cnt_ref[...] += cnt

        @pl.when(t == tsteps - 1)
        def _fin():
            denom = jnp.maximum(cnt_ref[:, 0:1], 1.0)
            pooled = pooled_acc[rows, :] / denom
            pooled_acc[rows, :] = pooled
            logits_ref[...] = lax.dot_general(
                pooled, rw_ref[...], (((1,), (1,)), ((), ())),
                preferred_element_type=jnp.float32)

    @pl.when(i >= npool)
    def _expert():
        e = i - npool
        bias = b_ref[...]
        sel = lax.broadcasted_iota(jnp.int32, bias.shape, 0) == e
        brow = jnp.sum(jnp.where(sel, bias, 0.0), axis=0, keepdims=True)
        y = lax.dot_general(
            pooled_acc[...], w_ref[0], (((1,), (1,)), ((), ())),
            preferred_element_type=jnp.float32) + brow
        pad = jnp.zeros((y.shape[0], CP - C), jnp.float32)
        y_ref[0] = jnp.concatenate([y, pad], axis=1)


def _pool_and_heads(x, mask, router_W, expert_W, expert_b, CP):
    B, T, D = x.shape
    E, C, _ = expert_W.shape
    BB, TB = 8, 128
    nb, ts = B // BB, T // TB
    npool = nb * ts
    grid = (npool + E,)

    def xmap(i):
        return (jnp.minimum(i // ts, nb - 1),
                jnp.where(i < npool, i % ts, ts - 1), 0)

    logits, y = pl.pallas_call(
        functools.partial(_fused_body, bb=BB, tsteps=ts, npool=npool,
                          E=E, C=C, CP=CP),
        grid=grid,
        in_specs=[
            pl.BlockSpec((BB, TB), lambda i: (jnp.minimum(i // ts, nb - 1),
                                              jnp.where(i < npool, i % ts,
                                                        ts - 1))),
            pl.BlockSpec((E, D), lambda i: (0, 0)),
            pl.BlockSpec((E, C), lambda i: (0, 0)),
            pl.BlockSpec((BB, TB, D), xmap),
            pl.BlockSpec((1, C, D), lambda i: (jnp.maximum(i - npool, 0), 0, 0)),
        ],
        out_specs=[
            pl.BlockSpec((BB, E), lambda i: (jnp.minimum(i // ts, nb - 1), 0)),
            pl.BlockSpec((1, B, CP), lambda i: (jnp.maximum(i - npool, 0), 0, 0)),
        ],
        out_shape=[
            jax.ShapeDtypeStruct((B, E), jnp.float32),
            jax.ShapeDtypeStruct((E, B, CP), jnp.float32),
        ],
        scratch_shapes=[
            pltpu.VMEM((B, D), jnp.float32),
            pltpu.VMEM((BB, 128), jnp.float32),
        ],
        compiler_params=pltpu.CompilerParams(
            dimension_semantics=("arbitrary",)),
    )(mask, router_W, expert_b, x, expert_W)
    return logits, y


def _lane_perm(v, idx):
    return lax.gather(
        v, idx[:, None],
        lax.GatherDimensionNumbers(
            offset_dims=(), collapsed_slice_dims=(0,), start_index_map=(0,)),
        slice_sizes=(1,),
        mode=lax.GatherScatterMode.PROMISE_IN_BOUNDS)


def _butterfly(v, iota, op):
    for k in (1, 2, 4, 8):
        v = op(v, _lane_perm(v, iota ^ k))
    return v


def _make_route_combine(B, E, CP):
    info = plsc.get_sparse_core_info()
    nw = info.num_cores * info.num_subcores
    rows = B // nw
    nch = CP // 16

    @functools.partial(
        pl.kernel,
        mesh=plsc.VectorSubcoreMesh(core_axis_name="c", subcore_axis_name="s"),
        out_type=jax.ShapeDtypeStruct((B, CP), jnp.float32),
        scratch_types=[
            pltpu.VMEM((rows, E), jnp.float32),
            pltpu.VMEM((16,), jnp.int32),
            pltpu.VMEM((16, CP), jnp.float32),
            pltpu.VMEM((rows, CP), jnp.float32),
            pltpu.SemaphoreType.DMA,
        ],
    )
    def route_combine(logits_hbm, y_hbm, out_hbm, logit_v, idx_v, rows_v,
                      out_v, sem):
        wid = lax.axis_index("s") * info.num_cores + lax.axis_index("c")
        base = wid * rows
        pltpu.sync_copy(logits_hbm.at[pl.ds(base, rows)], logit_v)
        iota = lax.iota(jnp.int32, E)
        big = jnp.int32(E)
        gates = []
        idx_vec = jnp.zeros((16,), jnp.int32)
        for i in range(rows):
            row = logit_v[i, :]
            m = _butterfly(row, iota, jnp.maximum)
            p = jnp.exp(row - m)
            z = _butterfly(p, iota, jnp.add)
            probs = p / z
            v1 = _butterfly(probs, iota, jnp.maximum)
            i1 = _butterfly(jnp.where(probs == v1, iota, big), iota,
                            jnp.minimum)
            rest = jnp.where(iota == i1, -1.0, probs)
            v2 = _butterfly(rest, iota, jnp.maximum)
            i2 = _butterfly(jnp.where(rest == v2, iota, big), iota,
                            jnp.minimum)
            denom = v1 + v2 + 1e-9
            gates.append((v1 / denom, v2 / denom))
            tok = base + i
            idx_vec = jnp.where(iota == 2 * i, i1 * B + tok, idx_vec)
            idx_vec = jnp.where(iota == 2 * i + 1, i2 * B + tok, idx_vec)
        idx_v[...] = idx_vec
        pltpu.async_copy(y_hbm.at[idx_v], rows_v, sem).wait()
        for i in range(rows):
            g1, g2 = gates[i]
            for j in range(nch):
                sl = pl.ds(j * 16, 16)
                out_v[i, sl] = g1 * rows_v[2 * i, sl] + g2 * rows_v[2 * i + 1, sl]
        pltpu.sync_copy(out_v, out_hbm.at[pl.ds(base, rows)])

    return route_combine


def kernel(x, mask, router_W, expert_W, expert_b):
    B = x.shape[0]
    E, C, _ = expert_W.shape
    CP = 1024
    logits, y = _pool_and_heads(x, mask, router_W, expert_W, expert_b, CP)
    out_p = _make_route_combine(B, E, CP)(logits, y.reshape(E * B, CP))
    return out_p[:, :C]


# speedup vs baseline: 1.0696x; 1.0696x over previous
import functools

import jax
import jax.numpy as jnp
from jax import lax
from jax.experimental import pallas as pl
from jax.experimental.pallas import tpu as pltpu
from jax.experimental.pallas import tpu_sc as plsc


def _pool_body(mask_ref, rw_ref, x_ref, pooled_ref, logits_ref, cnt_ref, *,
               t_blocks):
    t = pl.program_id(1)

    @pl.when(t == 0)
    def _init():
        pooled_ref[...] = jnp.zeros_like(pooled_ref)
        cnt_ref[...] = jnp.zeros_like(cnt_ref)

    mask_f = mask_ref[...].astype(jnp.float32)
    x = x_ref[...]
    pooled_ref[...] += jnp.sum(x * mask_f[:, :, None], axis=1)
    cnt_ref[...] += jnp.sum(mask_f, axis=1, keepdims=True)

    @pl.when(t == t_blocks - 1)
    def _fin():
        denom = jnp.maximum(cnt_ref[:, 0:1], 1.0)
        pooled = pooled_ref[...] / denom
        pooled_ref[...] = pooled
        lg = lax.dot_general(
            pooled, rw_ref[...], (((1,), (1,)), ((), ())),
            preferred_element_type=jnp.float32)
        pad = jnp.zeros((lg.shape[0], 128 - lg.shape[1]), jnp.float32)
        logits_ref[...] = jnp.concatenate([lg, pad], axis=1)


def _pool_and_route(x, mask, router_W):
    B, T, D = x.shape
    E = router_W.shape[0]
    BB, TB = 8, 256
    grid = (B // BB, T // TB)
    return pl.pallas_call(
        functools.partial(_pool_body, t_blocks=grid[1]),
        grid=grid,
        in_specs=[
            pl.BlockSpec((BB, TB), lambda b, t: (b, t)),
            pl.BlockSpec((E, D), lambda b, t: (0, 0)),
            pl.BlockSpec((BB, TB, D), lambda b, t: (b, t, 0)),
        ],
        scratch_shapes=[pltpu.VMEM((BB, 128), jnp.float32)],
        out_specs=[
            pl.BlockSpec((BB, D), lambda b, t: (b, 0)),
            pl.BlockSpec((BB, 128), lambda b, t: (b, 0)),
        ],
        out_shape=[
            jax.ShapeDtypeStruct((B, D), jnp.float32),
            jax.ShapeDtypeStruct((B, 128), jnp.float32),
        ],
        compiler_params=pltpu.CompilerParams(
            dimension_semantics=("parallel", "arbitrary")),
    )(mask, router_W, x)


def _lane_perm(v, idx):
    return lax.gather(
        v, idx[:, None],
        lax.GatherDimensionNumbers(
            offset_dims=(), collapsed_slice_dims=(0,), start_index_map=(0,)),
        slice_sizes=(1,),
        mode=lax.GatherScatterMode.PROMISE_IN_BOUNDS)


def _butterfly(v, iota, op):
    for k in (1, 2, 4, 8):
        v = op(v, _lane_perm(v, iota ^ k))
    return v


def _make_gates_kernel(B, E):
    info = plsc.get_sparse_core_info()
    nw = info.num_cores * info.num_subcores
    rows = B // nw

    @functools.partial(
        pl.kernel,
        mesh=plsc.VectorSubcoreMesh(core_axis_name="c", subcore_axis_name="s"),
        out_type=jax.ShapeDtypeStruct((B, 128), jnp.float32),
        scratch_types=[
            pltpu.VMEM((rows, 128), jnp.float32),
            pltpu.VMEM((rows, 128), jnp.float32),
        ],
    )
    def gates_kernel(logits_hbm, out_hbm, in_v, out_v):
        wid = lax.axis_index("s") * info.num_cores + lax.axis_index("c")
        base = wid * rows
        pltpu.sync_copy(logits_hbm.at[pl.ds(base, rows)], in_v)
        iota = lax.iota(jnp.int32, E)
        big = jnp.int32(E)
        zeros16 = jnp.zeros((16,), jnp.float32)
        for i in range(rows):
            row = in_v[i, pl.ds(0, E)]
            m = _butterfly(row, iota, jnp.maximum)
            p = jnp.exp(row - m)
            z = _butterfly(p, iota, jnp.add)
            probs = p / z
            v1 = _butterfly(probs, iota, jnp.maximum)
            i1 = _butterfly(jnp.where(probs == v1, iota, big), iota,
                            jnp.minimum)
            rest = jnp.where(iota == i1, -1.0, probs)
            v2 = _butterfly(rest, iota, jnp.maximum)
            i2 = _butterfly(jnp.where(rest == v2, iota, big), iota,
                            jnp.minimum)
            denom = v1 + v2 + 1e-9
            g = jnp.where(iota == i1, v1 / denom,
                          jnp.where(iota == i2, v2 / denom, 0.0))
            out_v[i, pl.ds(0, E)] = g
            for c in range(1, 8):
                out_v[i, pl.ds(16 * c, 16)] = zeros16
        pltpu.sync_copy(out_v, out_hbm.at[pl.ds(base, rows)])

    return gates_kernel


def _expert_body(gates_ref, b_ref, pooled_ref, w_ref, out_ref, *, epb):
    e = pl.program_id(0)
    gates = gates_ref[...]

    @pl.when(e == 0)
    def _init():
        out_ref[...] = jnp.dot(gates[:, :b_ref.shape[0]], b_ref[...],
                               preferred_element_type=jnp.float32)

    eidx = lax.broadcasted_iota(jnp.int32, gates.shape, 1)
    acc = out_ref[...]
    for j in range(epb):
        sel = eidx == (e * epb + j)
        col = jnp.sum(jnp.where(sel, gates, 0.0), axis=1, keepdims=True)
        y = lax.dot_general(
            pooled_ref[...], w_ref[j],
            (((1,), (1,)), ((), ())),
            preferred_element_type=jnp.float32)
        acc = acc + y * col
    out_ref[...] = acc


def _expert_combine(gates, expert_b, pooled, expert_W):
    E, C, D = expert_W.shape
    B = pooled.shape[0]
    EPB = 2
    return pl.pallas_call(
        functools.partial(_expert_body, epb=EPB),
        grid=(E // EPB,),
        in_specs=[
            pl.BlockSpec((B, 128), lambda e: (0, 0)),
            pl.BlockSpec((E, C), lambda e: (0, 0)),
            pl.BlockSpec((B, D), lambda e: (0, 0)),
            pl.BlockSpec((EPB, C, D), lambda e: (e, 0, 0)),
        ],
        out_specs=pl.BlockSpec((B, C), lambda e: (0, 0)),
        out_shape=jax.ShapeDtypeStruct((B, C), jnp.float32),
        compiler_params=pltpu.CompilerParams(
            dimension_semantics=("arbitrary",)),
    )(gates, expert_b, pooled, expert_W)


def kernel(x, mask, router_W, expert_W, expert_b):
    pooled, logits = _pool_and_route(x, mask, router_W)
    gates = _make_gates_kernel(x.shape[0], router_W.shape[0])(logits)
    return _expert_combine(gates, expert_b, pooled, expert_W)
